# baseline (speedup 1.0000x reference)
import functools

import jax
import jax.numpy as jnp
from jax import lax
from jax.experimental import pallas as pl
from jax.experimental.pallas import tpu as pltpu
from jax.experimental.pallas import tpu_sc as plsc

_EMBED = 64
_SCALE = 8.0
_NC, _NS, _L = 2, 16, 16
_NW = _NC * _NS

_BC = 512
_KI = _BC // 128


def _scale_body(t_ref, o_ref):
    o_ref[...] = t_ref[...] * _SCALE


def _scaled_table(table):
    return pl.pallas_call(
        _scale_body,
        out_shape=jax.ShapeDtypeStruct(table.shape, table.dtype),
    )(table)


@functools.cache
def _gather_kernel(n_rows):
    per_w = n_rows // _NW
    chunks = per_w // _BC
    assert per_w % _BC == 0 and n_rows % _NW == 0

    def body(x_hbm, tbl_hbm, out_hbm, xv, idxv, rowsv, gsem):
        wid = lax.axis_index("s") * _NC + lax.axis_index("c")
        w0 = wid * per_w
        strided = lax.iota(jnp.int32, _L) * 3

        def chunk(c, carry):
            base = w0 + c * _BC
            pltpu.sync_copy(x_hbm.at[pl.ds(base * 3, _BC * 3)], xv)
            for m in range(_BC // _L):
                v = plsc.load_gather(xv, [strided + (m * _L * 3)])
                idxv[m // 8, pl.ds((m % 8) * _L, _L)] = v
            copies = [
                pltpu.async_copy(
                    tbl_hbm.at[idxv.at[j]],
                    rowsv.at[pl.ds(j * 128, 128)],
                    gsem,
                )
                for j in range(_KI)
            ]
            for cp in copies:
                cp.wait()
            pltpu.sync_copy(rowsv, out_hbm.at[pl.ds(base, _BC)])
            return carry

        lax.fori_loop(0, chunks, chunk, 0)

    return pl.kernel(
        body,
        out_type=jax.ShapeDtypeStruct((n_rows, _EMBED), jnp.float32),
        mesh=plsc.VectorSubcoreMesh(core_axis_name="c", subcore_axis_name="s"),
        compiler_params=pltpu.CompilerParams(
            needs_layout_passes=False, use_tc_tiling_on_sc=False
        ),
        scratch_types=[
            pltpu.VMEM((_BC * 3,), jnp.int32),
            pltpu.VMEM((_KI, 128), jnp.int32),
            pltpu.VMEM((_BC, _EMBED), jnp.float32),
            pltpu.SemaphoreType.DMA,
        ],
    )


def kernel(x, table):
    b, t, _ = x.shape
    n = b * t
    x2 = x.reshape(n * 3)
    tbl = _scaled_table(table)
    out = _gather_kernel(n)(x2, tbl)
    return out.reshape(b, t, _EMBED)


# speedup vs baseline: 1.4264x; 1.4264x over previous
import functools

import jax
import jax.numpy as jnp
from jax import lax
from jax.experimental import pallas as pl
from jax.experimental.pallas import tpu as pltpu
from jax.experimental.pallas import tpu_sc as plsc

_EMBED = 64
_SCALE = 8.0
_V = 23
_NC, _NS, _L = 2, 16, 16
_NW = _NC * _NS

_PBC = 256
_KI = _PBC // 128


def _pair_table_body(t_ref, o_ref):
    t = t_ref[...] * _SCALE
    a = jnp.broadcast_to(t[:, None, :], (_V, _V, _EMBED)).reshape(_V * _V, _EMBED)
    b = jnp.broadcast_to(t[None, :, :], (_V, _V, _EMBED)).reshape(_V * _V, _EMBED)
    o_ref[...] = jnp.concatenate([a, b], axis=1)


def _pair_table(table):
    return pl.pallas_call(
        _pair_table_body,
        out_shape=jax.ShapeDtypeStruct((_V * _V, 2 * _EMBED), jnp.float32),
    )(table)


@functools.cache
def _gather_kernel(n_pairs):
    per_w = n_pairs // _NW
    chunks = per_w // _PBC
    assert per_w % _PBC == 0 and n_pairs % _NW == 0

    def body(x_hbm, tbl_hbm, out_hbm, xv, idxv, rowsv, gsem):
        wid = lax.axis_index("s") * _NC + lax.axis_index("c")
        w0 = wid * per_w
        pos_a = lax.iota(jnp.int32, _L) * 6

        def chunk(c, carry):
            pbase = w0 + c * _PBC
            pltpu.sync_copy(x_hbm.at[pl.ds(pbase * 6, _PBC * 6)], xv)
            for m in range(_PBC // _L):
                va = plsc.load_gather(xv, [pos_a + (m * _L * 6)])
                vb = plsc.load_gather(xv, [pos_a + (m * _L * 6 + 3)])
                idxv[m // 8, pl.ds((m % 8) * _L, _L)] = va * _V + vb
            copies = [
                pltpu.async_copy(
                    tbl_hbm.at[idxv.at[j]],
                    rowsv.at[pl.ds(j * 128, 128)],
                    gsem,
                )
                for j in range(_KI)
            ]
            for cp in copies:
                cp.wait()
            pltpu.sync_copy(rowsv, out_hbm.at[pl.ds(pbase, _PBC)])
            return carry

        lax.fori_loop(0, chunks, chunk, 0)

    return pl.kernel(
        body,
        out_type=jax.ShapeDtypeStruct((n_pairs, 2 * _EMBED), jnp.float32),
        mesh=plsc.VectorSubcoreMesh(core_axis_name="c", subcore_axis_name="s"),
        compiler_params=pltpu.CompilerParams(needs_layout_passes=False),
        scratch_types=[
            pltpu.VMEM((_PBC * 6,), jnp.int32),
            pltpu.VMEM((_KI, 128), jnp.int32),
            pltpu.VMEM((_PBC, 2 * _EMBED), jnp.float32),
            pltpu.SemaphoreType.DMA,
        ],
    )


def kernel(x, table):
    b, t, _ = x.shape
    n = b * t
    x2 = x.reshape(n * 3)
    tbl = _pair_table(table)
    out = _gather_kernel(n // 2)(x2, tbl)
    return out.reshape(b, t, _EMBED)


# speedup vs baseline: 68.0267x; 47.6915x over previous
import functools

import jax
import jax.numpy as jnp
from jax import lax
from jax.experimental import pallas as pl
from jax.experimental.pallas import tpu as pltpu

_EMBED = 64
_SCALE = 8.0
_V = 23

_BJ = 8
_BL = 2048


def _onehot_body(x_ref, t_ref, o_ref):
    t8 = t_ref[...] * _SCALE
    vals = lax.broadcasted_iota(jnp.int32, (_V, _BL), 0)
    for jj in range(_BJ):
        idx = x_ref[0, jj, :]
        oh = (idx[None, :] == vals).astype(jnp.float32)
        o_ref[jj] = lax.dot_general(
            t8, oh, (((0,), (0,)), ((), ())),
            preferred_element_type=jnp.float32,
        )


@functools.cache
def _lookup_kernel(nj, ni):
    grid = (nj // _BJ, ni // _BL)
    return pl.pallas_call(
        _onehot_body,
        grid=grid,
        in_specs=[
            pl.BlockSpec((1, _BJ, _BL), lambda j, i: (0, j, i)),
            pl.BlockSpec((_V, _EMBED), lambda j, i: (0, 0)),
        ],
        out_specs=pl.BlockSpec((_BJ, _EMBED, _BL), lambda j, i: (j, 0, i)),
        out_shape=jax.ShapeDtypeStruct((nj, _EMBED, ni), jnp.float32),
    )


def kernel(x, table):
    b, t, _ = x.shape
    xt = jnp.transpose(x, (2, 1, 0))
    outt = _lookup_kernel(t, b)(xt, table)
    return jnp.transpose(outt, (2, 0, 1))
